# Initial kernel scaffold; baseline (speedup 1.0000x reference)
import jax
import jax.numpy as jnp
from jax.experimental import pallas as pl


def kernel(x, edge_index, Wskip, bskip, Wl1, Wr1, att1, bias1, g1, be1, Wl2, Wr2, att2, bias2, g2, be2, Wlin, blin):
    raise NotImplementedError("write your pallas kernel here")


# speedup vs baseline: 11.4094x; 11.4094x over previous
import functools

import jax
import jax.numpy as jnp
from jax import lax
from jax.experimental import pallas as pl
from jax.experimental.pallas import tpu as pltpu
from jax.experimental.pallas import tpu_sc as plsc

N = 10000
E = 320000
IN_DIM = 128
HID = 64
HEADS = 4

NP = 10240
DUMMY = N
B = 128
EP = 323584
TILE_E1 = EP // 16
NB1 = TILE_E1 // B
TILE_E2 = EP // 32
NB2 = TILE_E2 // B
RPT = NP // 16
BLK = 512
NBLK = NP // BLK
HROW = 5120
ACCR = HROW + 8
RPH = HROW // 16

_mesh = plsc.VectorSubcoreMesh(
    core_axis_name="c", subcore_axis_name="s", num_cores=2, num_subcores=16)


def _phase_a_body(x_ref, wl_ref, wr_ref, att_ref, wsk_ref, bsk_ref,
                  xl_out, xr_out, m_out, xsk_out):
    xb = x_ref[...]
    xl = jnp.dot(xb, wl_ref[...], preferred_element_type=jnp.float32)
    xr = jnp.dot(xb, wr_ref[...], preferred_element_type=jnp.float32)
    z = xl + xr
    e = jnp.maximum(z, 0.2 * z)
    ea = e * att_ref[...].reshape(1, 2 * HID)
    col = lax.broadcasted_iota(jnp.int32, (1, 2 * HID), 1)
    m0 = jnp.sum(jnp.where(col < HID, ea, 0.0), axis=1)
    m1 = jnp.sum(jnp.where(col >= HID, ea, 0.0), axis=1)
    lane = lax.broadcasted_iota(jnp.int32, (1, 16), 1)
    tail = (m0[:, None] * (lane == 0).astype(jnp.float32)
            + m1[:, None] * (lane == 1).astype(jnp.float32))
    xl_out[...] = xl[None]
    xr_out[...] = xr[None]
    m_out[...] = tail[None]
    xsk_out[...] = jnp.dot(xb, wsk_ref[...],
                           preferred_element_type=jnp.float32) + bsk_ref[...]


def _phase_a(x_p, Wl1, Wr1, att1r, Wskip, bskip):
    return pl.pallas_call(
        _phase_a_body,
        grid=(2, NBLK),
        in_specs=[
            pl.BlockSpec((BLK, IN_DIM), lambda s, i: (i, 0)),
            pl.BlockSpec((IN_DIM, 2 * HID), lambda s, i: (0, s)),
            pl.BlockSpec((IN_DIM, 2 * HID), lambda s, i: (0, s)),
            pl.BlockSpec((1, 1, 2 * HID), lambda s, i: (s, 0, 0)),
            pl.BlockSpec((IN_DIM, HID), lambda s, i: (0, 0)),
            pl.BlockSpec((1, HID), lambda s, i: (0, 0)),
        ],
        out_specs=[
            pl.BlockSpec((1, BLK, 2 * HID), lambda s, i: (s, i, 0)),
            pl.BlockSpec((1, BLK, 2 * HID), lambda s, i: (s, i, 0)),
            pl.BlockSpec((1, BLK, 16), lambda s, i: (s, i, 0)),
            pl.BlockSpec((BLK, HID), lambda s, i: (i, 0)),
        ],
        out_shape=[
            jax.ShapeDtypeStruct((2, NP, 2 * HID), jnp.float32),
            jax.ShapeDtypeStruct((2, NP, 2 * HID), jnp.float32),
            jax.ShapeDtypeStruct((2, NP, 16), jnp.float32),
            jax.ShapeDtypeStruct((NP, HID), jnp.float32),
        ],
    )(x_p, Wl1, Wr1, att1r, Wskip, bskip)


def _l1_body(xl_hbm, xr_hbm, att_hbm, srcp_hbm, dstp_hbm, dst_hbm,
             dl0_hbm, dl1_hbm, out_hbm, outden_hbm, srcv, dstva, dstvp,
             dstvl, xlbuf, xrbuf, comb, attv, den, acc, sem1, sem2):
    cid = lax.axis_index("c")
    sid = lax.axis_index("s")
    zv = jnp.zeros((16,), jnp.float32)
    lane = lax.iota(jnp.int32, 16)

    def zero_comb(b, c):
        for k in range(8):
            comb[b, pl.ds(16 * k, 16)] = zv
        return c
    lax.fori_loop(0, B, zero_comb, 0)

    def zero_den(j, c):
        den[pl.ds(16 * j, 16)] = zv
        return c
    lax.fori_loop(0, 2 * NP // 16, zero_den, 0)

    r0 = sid * RPH
    pltpu.sync_copy(att_hbm.at[cid], attv)
    atts = [attv[pl.ds(16 * k, 16)] for k in range(8)]

    ebase = cid * EP + sid * TILE_E1
    pbase = sid * TILE_E1

    for p in range(2):
        for j in range(2):
            pltpu.sync_copy(comb, acc.at[pl.ds(r0 + j * B, B)])
        pltpu.sync_copy(comb.at[pl.ds(0, 64)], acc.at[pl.ds(r0 + 2 * B, 64)])
        plsc.subcore_barrier()

        dl_hbm = dl0_hbm if p == 0 else dl1_hbm

        def batch_body(g, c):
            off = ebase + g * B
            poff = pbase + g * B
            pltpu.sync_copy(srcp_hbm.at[pl.ds(off, B)], srcv)
            pltpu.sync_copy(dstp_hbm.at[pl.ds(off, B)], dstva)
            pltpu.sync_copy(dst_hbm.at[pl.ds(poff, B)], dstvp)
            pltpu.sync_copy(dl_hbm.at[pl.ds(poff, B)], dstvl)
            cp1 = pltpu.async_copy(xl_hbm.at[srcv], xlbuf, sem1)
            cp2 = pltpu.async_copy(xr_hbm.at[dstva], xrbuf, sem2)
            cp1.wait()
            cp2.wait()

            def edge(b, cc):
                svs = [xlbuf[b, pl.ds(16 * k, 16)] for k in range(8)]
                rvs = [xrbuf[b, pl.ds(16 * k, 16)] for k in range(8)]
                ts = []
                for k in range(8):
                    zz = svs[k] + rvs[k]
                    ts.append(jnp.maximum(zz, 0.2 * zz) * atts[k])
                u0 = (ts[0] + ts[1]) + (ts[2] + ts[3])
                u1 = (ts[4] + ts[5]) + (ts[6] + ts[7])
                l0 = jnp.sum(u0)
                l1 = jnp.sum(u1)
                w0 = jnp.exp(jnp.full((16,), l0, jnp.float32))
                w1 = jnp.exp(jnp.full((16,), l1, jnp.float32))
                for k in range(4):
                    comb[b, pl.ds(16 * k, 16)] = svs[k] * w0
                for k in range(4, 8):
                    comb[b, pl.ds(16 * k, 16)] = svs[k] * w1
                if p == 0:
                    bbase = (b >> 4) << 4
                    lm = lane == (b & 15)
                    dv = dstvp[pl.ds(bbase, 16)]
                    plsc.addupdate_scatter(den, [dv], w0, mask=lm)
                    plsc.addupdate_scatter(den, [dv + NP], w1, mask=lm)
                return cc
            lax.fori_loop(0, B, edge, 0)
            pltpu.sync_copy(comb, acc.at[dstvl], add=True)
            return c
        lax.fori_loop(0, NB1, batch_body, 0)
        plsc.subcore_barrier()

        outoff = cid * NP + p * HROW + r0
        for j in range(2):
            pltpu.sync_copy(acc.at[pl.ds(r0 + j * B, B)],
                            out_hbm.at[pl.ds(outoff + j * B, B)])
        pltpu.sync_copy(acc.at[pl.ds(r0 + 2 * B, 64)],
                        out_hbm.at[pl.ds(outoff + 2 * B, 64)])
        if p == 0:
            lax.fori_loop(0, B, zero_comb, 0)
    pltpu.sync_copy(den, outden_hbm.at[cid * 16 + sid])


_l1_edges = functools.partial(
    pl.kernel, _l1_body,
    out_type=[
        jax.ShapeDtypeStruct((2 * NP, 2 * HID), jnp.float32),
        jax.ShapeDtypeStruct((32, 2 * NP), jnp.float32),
    ],
    mesh=_mesh,
    compiler_params=pltpu.CompilerParams(needs_layout_passes=False),
    scratch_types=[
        pltpu.VMEM((B,), jnp.int32),
        pltpu.VMEM((B,), jnp.int32),
        pltpu.VMEM((B,), jnp.int32),
        pltpu.VMEM((B,), jnp.int32),
        pltpu.VMEM((B, 2 * HID), jnp.float32),
        pltpu.VMEM((B, 2 * HID), jnp.float32),
        pltpu.VMEM((B, 2 * HID), jnp.float32),
        pltpu.VMEM((2 * HID,), jnp.float32),
        pltpu.VMEM((2 * NP,), jnp.float32),
        pltpu.VMEM_SHARED((ACCR, 2 * HID), jnp.float32),
        pltpu.SemaphoreType.DMA,
        pltpu.SemaphoreType.DMA,
    ])()


def _phase_c1_body(xl_ref, acc_ref, den_ref, m_ref, b1_ref, h_out, sums_out):
    i = pl.program_id(0)
    accv = acc_ref[...]
    denv = den_ref[...]
    mv = m_ref[...]
    colh = lax.broadcasted_iota(jnp.int32, (1, 2 * HID), 1)
    lane = lax.broadcasted_iota(jnp.int32, (1, 16), 1)
    blks = []
    for s in range(2):
        dsc = jnp.sum(lax.slice(denv, (16 * s, 0, 0), (16 * s + 16, 2, BLK)),
                      axis=0)
        em0 = jnp.exp(jnp.sum(jnp.where(lane == 0, mv[s], 0.0), axis=1))
        em1 = jnp.exp(jnp.sum(jnp.where(lane == 1, mv[s], 0.0), axis=1))
        denb = jnp.where(colh < HID, dsc[0][:, None], dsc[1][:, None])
        emb = jnp.where(colh < HID, em0[:, None], em1[:, None])
        blks.append((xl_ref[s] * emb + accv[s]) / (emb + denb)
                    + b1_ref[s][None])
    h = jnp.stack(blks, axis=0)
    h_out[...] = h
    row = lax.broadcasted_iota(jnp.int32, (BLK, 1), 0) + i * BLK
    hm = jnp.where((row < N)[None], h, 0.0)
    part = jnp.stack([jnp.sum(hm, axis=1), jnp.sum(hm * hm, axis=1)], axis=0)

    @pl.when(i == 0)
    def _():
        sums_out[...] = part

    @pl.when(i > 0)
    def _():
        sums_out[...] += part


def _phase_c1(XL1, ACC1, DEN1, M1, bias1):
    return pl.pallas_call(
        _phase_c1_body,
        grid=(NBLK,),
        in_specs=[
            pl.BlockSpec((2, BLK, 2 * HID), lambda i: (0, i, 0)),
            pl.BlockSpec((2, BLK, 2 * HID), lambda i: (0, i, 0)),
            pl.BlockSpec((32, 2, BLK), lambda i: (0, 0, i)),
            pl.BlockSpec((2, BLK, 16), lambda i: (0, i, 0)),
            pl.BlockSpec((2, 2 * HID), lambda i: (0, 0)),
        ],
        out_specs=[
            pl.BlockSpec((2, BLK, 2 * HID), lambda i: (0, i, 0)),
            pl.BlockSpec((2, 2, 2 * HID), lambda i: (0, 0, 0)),
        ],
        out_shape=[
            jax.ShapeDtypeStruct((2, NP, 2 * HID), jnp.float32),
            jax.ShapeDtypeStruct((2, 2, 2 * HID), jnp.float32),
        ],
    )(XL1, ACC1, DEN1, M1, bias1)


def _phase_c2_body(h_ref, sums_ref, g_ref, be_ref, wl_ref, wr_ref, wlr_ref,
                   att_ref, xlr_out, m_out):
    sums = sums_ref[...]
    mu = sums[0] / N
    var = sums[1] / N - mu * mu
    A = g_ref[...] * lax.rsqrt(var + 1e-5)
    Bc = be_ref[...] - mu * A
    h = h_ref[...]
    hn = jnp.maximum(h * A[:, None, :] + Bc[:, None, :], 0.0)
    xl2 = (jnp.dot(hn[0], wl_ref[0], preferred_element_type=jnp.float32)
           + jnp.dot(hn[1], wl_ref[1], preferred_element_type=jnp.float32))
    xr2 = (jnp.dot(hn[0], wr_ref[0], preferred_element_type=jnp.float32)
           + jnp.dot(hn[1], wr_ref[1], preferred_element_type=jnp.float32))
    xlr_out[...] = (jnp.dot(hn[0], wlr_ref[0], preferred_element_type=jnp.float32)
                    + jnp.dot(hn[1], wlr_ref[1], preferred_element_type=jnp.float32))
    z = xl2 + xr2
    e = jnp.maximum(z, 0.2 * z)
    m2 = jnp.sum(e * att_ref[...], axis=1)
    lane = lax.broadcasted_iota(jnp.int32, (1, 16), 1)
    m_out[...] = m2[:, None] * (lane == 0).astype(jnp.float32)


def _phase_c2(H1, sums1, g1, be1, Wl2, Wr2, Wlr2, att2):
    return pl.pallas_call(
        _phase_c2_body,
        grid=(NBLK,),
        in_specs=[
            pl.BlockSpec((2, BLK, 2 * HID), lambda i: (0, i, 0)),
            pl.BlockSpec((2, 2, 2 * HID), lambda i: (0, 0, 0)),
            pl.BlockSpec((2, 2 * HID), lambda i: (0, 0)),
            pl.BlockSpec((2, 2 * HID), lambda i: (0, 0)),
            pl.BlockSpec((2, 2 * HID, HID), lambda i: (0, 0, 0)),
            pl.BlockSpec((2, 2 * HID, HID), lambda i: (0, 0, 0)),
            pl.BlockSpec((2, 2 * HID, 2 * HID), lambda i: (0, 0, 0)),
            pl.BlockSpec((1, HID), lambda i: (0, 0)),
        ],
        out_specs=[
            pl.BlockSpec((BLK, 2 * HID), lambda i: (i, 0)),
            pl.BlockSpec((BLK, 16), lambda i: (i, 0)),
        ],
        out_shape=[
            jax.ShapeDtypeStruct((NP, 2 * HID), jnp.float32),
            jax.ShapeDtypeStruct((NP, 16), jnp.float32),
        ],
    )(H1, sums1, g1, be1, Wl2, Wr2, Wlr2, att2)


def _l2_body(xlr_hbm, att_hbm, src_hbm, dst_hbm, dl0_hbm, dl1_hbm,
             out_hbm, srcv, dstv, dstvl, xlbuf, xrbuf, comb, attv, acc,
             sem1, sem2):
    cid = lax.axis_index("c")
    sid = lax.axis_index("s")
    zv = jnp.zeros((16,), jnp.float32)
    lane = lax.iota(jnp.int32, 16)

    def zero_comb(b, c):
        for k in range(8):
            comb[b, pl.ds(16 * k, 16)] = zv
        return c
    lax.fori_loop(0, B, zero_comb, 0)

    r0 = sid * RPH
    pltpu.sync_copy(att_hbm, attv)
    atts = [attv[pl.ds(16 * k, 16)] for k in range(4)]

    ebase = (sid * 2 + cid) * TILE_E2

    for p in range(2):
        for j in range(2):
            pltpu.sync_copy(comb, acc.at[pl.ds(r0 + j * B, B)])
        pltpu.sync_copy(comb.at[pl.ds(0, 64)], acc.at[pl.ds(r0 + 2 * B, 64)])
        plsc.subcore_barrier()

        dl_hbm = dl0_hbm if p == 0 else dl1_hbm

        def batch_body(g, c):
            off = ebase + g * B
            pltpu.sync_copy(src_hbm.at[pl.ds(off, B)], srcv)
            pltpu.sync_copy(dst_hbm.at[pl.ds(off, B)], dstv)
            pltpu.sync_copy(dl_hbm.at[pl.ds(off, B)], dstvl)
            cp1 = pltpu.async_copy(xlr_hbm.at[srcv], xlbuf, sem1)
            cp2 = pltpu.async_copy(xlr_hbm.at[dstv], xrbuf, sem2)
            cp1.wait()
            cp2.wait()

            def edge(b, cc):
                svs = [xlbuf[b, pl.ds(16 * k, 16)] for k in range(4)]
                rvs = [xrbuf[b, pl.ds(16 * k + 64, 16)] for k in range(4)]
                ts = []
                for k in range(4):
                    zz = svs[k] + rvs[k]
                    ts.append(jnp.maximum(zz, 0.2 * zz) * atts[k])
                u = (ts[0] + ts[1]) + (ts[2] + ts[3])
                l = jnp.sum(u)
                w = jnp.exp(jnp.full((16,), l, jnp.float32))
                for k in range(4):
                    comb[b, pl.ds(16 * k, 16)] = svs[k] * w
                comb[b, pl.ds(64, 16)] = jnp.where(lane == 0, w, 0.0)
                return cc
            lax.fori_loop(0, B, edge, 0)
            pltpu.sync_copy(comb, acc.at[dstvl], add=True)
            return c
        lax.fori_loop(0, NB2, batch_body, 0)
        plsc.subcore_barrier()

        outoff = cid * NP + p * HROW + r0
        for j in range(2):
            pltpu.sync_copy(acc.at[pl.ds(r0 + j * B, B)],
                            out_hbm.at[pl.ds(outoff + j * B, B)])
        pltpu.sync_copy(acc.at[pl.ds(r0 + 2 * B, 64)],
                        out_hbm.at[pl.ds(outoff + 2 * B, 64)])
        if p == 0:
            lax.fori_loop(0, B, zero_comb, 0)


_l2_edges = functools.partial(
    pl.kernel, _l2_body,
    out_type=jax.ShapeDtypeStruct((2 * NP, 2 * HID), jnp.float32),
    mesh=_mesh,
    compiler_params=pltpu.CompilerParams(needs_layout_passes=False),
    scratch_types=[
        pltpu.VMEM((B,), jnp.int32),
        pltpu.VMEM((B,), jnp.int32),
        pltpu.VMEM((B,), jnp.int32),
        pltpu.VMEM((B, 2 * HID), jnp.float32),
        pltpu.VMEM((B, 2 * HID), jnp.float32),
        pltpu.VMEM((B, 2 * HID), jnp.float32),
        pltpu.VMEM((HID,), jnp.float32),
        pltpu.VMEM_SHARED((ACCR, 2 * HID), jnp.float32),
        pltpu.SemaphoreType.DMA,
        pltpu.SemaphoreType.DMA,
    ])()


def _phase_e1_body(xlr_ref, acc_ref, m_ref, b2_ref, h_out, sums_out):
    i = pl.program_id(0)
    accv = acc_ref[...]
    lane = lax.broadcasted_iota(jnp.int32, (1, 16), 1)
    col = lax.broadcasted_iota(jnp.int32, (1, 2 * HID), 1)
    feat = (lax.slice(accv[0], (0, 0), (BLK, HID))
            + lax.slice(accv[1], (0, 0), (BLK, HID)))
    den = (jnp.sum(jnp.where(col == HID, accv[0], 0.0), axis=1)
           + jnp.sum(jnp.where(col == HID, accv[1], 0.0), axis=1))
    em = jnp.exp(jnp.sum(jnp.where(lane == 0, m_ref[...], 0.0), axis=1))
    xl2 = lax.slice(xlr_ref[...], (0, 0), (BLK, HID))
    h = ((xl2 * em[:, None] + feat) / (em + den)[:, None] + b2_ref[...])
    h_out[...] = h
    row = lax.broadcasted_iota(jnp.int32, (BLK, 1), 0) + i * BLK
    hm = jnp.where(row < N, h, 0.0)
    part = jnp.stack([jnp.sum(hm, axis=0), jnp.sum(hm * hm, axis=0)], axis=0)

    @pl.when(i == 0)
    def _():
        sums_out[...] = part

    @pl.when(i > 0)
    def _():
        sums_out[...] += part


def _phase_e1(XLR2, ACC2, M2, bias2):
    return pl.pallas_call(
        _phase_e1_body,
        grid=(NBLK,),
        in_specs=[
            pl.BlockSpec((BLK, 2 * HID), lambda i: (i, 0)),
            pl.BlockSpec((2, BLK, 2 * HID), lambda i: (0, i, 0)),
            pl.BlockSpec((BLK, 16), lambda i: (i, 0)),
            pl.BlockSpec((1, HID), lambda i: (0, 0)),
        ],
        out_specs=[
            pl.BlockSpec((BLK, HID), lambda i: (i, 0)),
            pl.BlockSpec((2, HID), lambda i: (0, 0)),
        ],
        out_shape=[
            jax.ShapeDtypeStruct((NP, HID), jnp.float32),
            jax.ShapeDtypeStruct((2, HID), jnp.float32),
        ],
    )(XLR2, ACC2, M2, bias2)


def _phase_e2_body(h_ref, sums_ref, g_ref, be_ref, xsk_ref, wlin_ref,
                   blin_ref, y_out):
    sums = sums_ref[...]
    mu = sums[0] / N
    var = sums[1] / N - mu * mu
    A = g_ref[...][0] * lax.rsqrt(var + 1e-5)
    Bc = be_ref[...][0] - mu * A
    hn = jnp.maximum(h_ref[...] * A[None] + Bc[None], 0.0)
    y_out[...] = jnp.dot(hn + xsk_ref[...], wlin_ref[...],
                         preferred_element_type=jnp.float32) + blin_ref[0, 0]


def _phase_e2(H2, sums2, g2, be2, XSK, Wlin_pad, blin):
    return pl.pallas_call(
        _phase_e2_body,
        grid=(NBLK,),
        in_specs=[
            pl.BlockSpec((BLK, HID), lambda i: (i, 0)),
            pl.BlockSpec((2, HID), lambda i: (0, 0)),
            pl.BlockSpec((1, HID), lambda i: (0, 0)),
            pl.BlockSpec((1, HID), lambda i: (0, 0)),
            pl.BlockSpec((BLK, HID), lambda i: (i, 0)),
            pl.BlockSpec((HID, 128), lambda i: (0, 0)),
            pl.BlockSpec((1, 1), lambda i: (0, 0)),
        ],
        out_specs=pl.BlockSpec((BLK, 128), lambda i: (i, 0)),
        out_shape=jax.ShapeDtypeStruct((NP, 128), jnp.float32),
    )(H2, sums2, g2, be2, XSK, Wlin_pad, blin)


def kernel(x, edge_index, Wskip, bskip, Wl1, Wr1, att1, bias1, g1, be1,
           Wl2, Wr2, att2, bias2, g2, be2, Wlin, blin):
    src = edge_index[0].astype(jnp.int32)
    dst = edge_index[1].astype(jnp.int32)
    padv = jnp.full((EP - E,), DUMMY, jnp.int32)
    src_p = jnp.concatenate([src, padv])
    dst_p = jnp.concatenate([dst, padv])
    src_pair = jnp.concatenate([src_p, src_p + NP])
    dst_pair = jnp.concatenate([dst_p, dst_p + NP])
    dl0 = jnp.where(dst_p < HROW, dst_p, HROW)
    d1 = dst_p - HROW
    dl1 = jnp.where(d1 >= 0, d1, HROW)
    x_p = jnp.zeros((NP, IN_DIM), jnp.float32).at[:N].set(x)

    att_sc = att1.reshape(2, 2 * HID)
    XL1, XR1, M1, XSK = _phase_a(x_p, Wl1, Wr1, att1.reshape(2, 1, 2 * HID),
                                 Wskip, bskip.reshape(1, HID))

    ACC1, DEN1 = _l1_edges(XL1.reshape(2 * NP, 2 * HID),
                           XR1.reshape(2 * NP, 2 * HID),
                           att_sc, src_pair, dst_pair, dst_p, dl0, dl1)

    H1, sums1 = _phase_c1(XL1, ACC1.reshape(2, NP, 2 * HID),
                          DEN1.reshape(32, 2, NP), M1,
                          bias1.reshape(2, 2 * HID))
    Wlr2 = jnp.concatenate([Wl2, Wr2], axis=1).reshape(2, 2 * HID, 2 * HID)
    XLR2, M2 = _phase_c2(H1, sums1, g1.reshape(2, 2 * HID),
                         be1.reshape(2, 2 * HID),
                         Wl2.reshape(2, 2 * HID, HID),
                         Wr2.reshape(2, 2 * HID, HID), Wlr2, att2)

    ACC2 = _l2_edges(XLR2, att2.reshape(HID), src_p, dst_p, dl0, dl1)

    H2, sums2 = _phase_e1(XLR2, ACC2.reshape(2, NP, 2 * HID), M2,
                          bias2.reshape(1, HID))
    Wlin_pad = jnp.zeros((HID, 128), jnp.float32).at[:, :1].set(Wlin)
    Y = _phase_e2(H2, sums2, g2.reshape(1, HID), be2.reshape(1, HID), XSK,
                  Wlin_pad, blin.reshape(1, 1))
    return Y[:N, 0]
